# baseline (speedup 1.0000x reference)
import jax
import jax.numpy as jnp
from jax.experimental import pallas as pl
from jax.experimental.pallas import tpu as pltpu


def _round_up(x, m):
    return (x + m - 1) // m * m


def _fused_kernel(g_ref, h_ref, wt_ref, b_ref, o_ref):
    t = jnp.dot(g_ref[...], h_ref[...], preferred_element_type=jnp.float32)
    o_ref[...] = jnp.maximum(
        jnp.dot(t, wt_ref[...], preferred_element_type=jnp.float32)
        + b_ref[...],
        0.0,
    ).astype(o_ref.dtype)


def kernel(g, h, w, b):
    n = g.shape[0]
    out_dim, in_dim = w.shape
    assert g.shape == (n, n) and h.shape == (n, in_dim)

    tm = 512 if n % 512 == 0 else 256
    n_pad = _round_up(n, tm)
    in_pad = _round_up(in_dim, 128)
    out_pad = _round_up(out_dim, 128)

    dtype = h.dtype
    g_p = jnp.pad(g, ((0, n_pad - n), (0, n_pad - n))).astype(dtype)
    h_p = jnp.pad(h, ((0, n_pad - n), (0, in_pad - in_dim))).astype(dtype)
    wt_p = jnp.pad(w.T, ((0, in_pad - in_dim), (0, out_pad - out_dim))).astype(dtype)
    b_p = jnp.pad(b.reshape(1, -1), ((0, 0), (0, out_pad - out_dim))).astype(jnp.float32)

    cost = pl.CostEstimate(
        flops=2 * n_pad * n_pad * in_pad + 2 * n_pad * in_pad * out_pad,
        transcendentals=0,
        bytes_accessed=4 * (n_pad * n_pad + n_pad * in_pad
                            + in_pad * out_pad + n_pad * out_pad),
    )
    out_p = pl.pallas_call(
        _fused_kernel,
        out_shape=jax.ShapeDtypeStruct((n_pad, out_pad), dtype),
        grid=(n_pad // tm,),
        in_specs=[
            pl.BlockSpec((tm, n_pad), lambda i: (i, 0)),
            pl.BlockSpec((n_pad, in_pad), lambda i: (0, 0)),
            pl.BlockSpec((in_pad, out_pad), lambda i: (0, 0)),
            pl.BlockSpec((1, out_pad), lambda i: (0, 0)),
        ],
        out_specs=pl.BlockSpec((tm, out_pad), lambda i: (i, 0)),
        compiler_params=pltpu.CompilerParams(
            dimension_semantics=("arbitrary",),
            vmem_limit_bytes=56 * 1024 * 1024,
        ),
        cost_estimate=cost,
    )(g_p, h_p, wt_p, b_p)

    return out_p[:n, :out_dim]


# speedup vs baseline: 1.0052x; 1.0052x over previous
import jax
import jax.numpy as jnp
from jax.experimental import pallas as pl
from jax.experimental.pallas import tpu as pltpu


def _round_up(x, m):
    return (x + m - 1) // m * m


def _fused_kernel(g_ref, h_ref, wt_ref, b_ref, o_ref):
    t = jnp.dot(g_ref[...], h_ref[...], preferred_element_type=jnp.float32)
    o_ref[...] = jnp.maximum(
        jnp.dot(t, wt_ref[...], preferred_element_type=jnp.float32)
        + b_ref[...],
        0.0,
    ).astype(o_ref.dtype)


def kernel(g, h, w, b):
    n = g.shape[0]
    out_dim, in_dim = w.shape
    assert g.shape == (n, n) and h.shape == (n, in_dim)

    tm = 512 if n % 512 == 0 else 256
    n_pad = _round_up(n, tm)
    in_pad = _round_up(in_dim, 128)
    out_pad = _round_up(out_dim, 128)

    dtype = h.dtype
    g_p = jnp.pad(g, ((0, n_pad - n), (0, n_pad - n))).astype(dtype)
    h_p = jnp.pad(h, ((0, n_pad - n), (0, in_pad - in_dim))).astype(dtype)
    wt_p = jnp.pad(w.T, ((0, in_pad - in_dim), (0, out_pad - out_dim))).astype(dtype)
    b_p = jnp.pad(b.reshape(1, -1), ((0, 0), (0, out_pad - out_dim))).astype(jnp.float32)

    cost = pl.CostEstimate(
        flops=2 * n_pad * n_pad * in_pad + 2 * n_pad * in_pad * out_pad,
        transcendentals=0,
        bytes_accessed=4 * (n_pad * n_pad + n_pad * in_pad
                            + in_pad * out_pad + n_pad * out_pad),
    )
    out_p = pl.pallas_call(
        _fused_kernel,
        out_shape=jax.ShapeDtypeStruct((n_pad, out_pad), dtype),
        grid=(n_pad // tm,),
        in_specs=[
            pl.BlockSpec((tm, n_pad), lambda i: (i, 0)),
            pl.BlockSpec((n_pad, in_pad), lambda i: (0, 0)),
            pl.BlockSpec((in_pad, out_pad), lambda i: (0, 0)),
            pl.BlockSpec((1, out_pad), lambda i: (0, 0)),
        ],
        out_specs=pl.BlockSpec((tm, out_pad), lambda i: (i, 0)),
        compiler_params=pltpu.CompilerParams(
            dimension_semantics=("parallel",),
            vmem_limit_bytes=56 * 1024 * 1024,
        ),
        cost_estimate=cost,
    )(g_p, h_p, wt_p, b_p)

    return out_p[:n, :out_dim]
